# Initial kernel scaffold; baseline (speedup 1.0000x reference)
import jax
import jax.numpy as jnp
from jax.experimental import pallas as pl


def kernel(g, edge_index, W1_root, W1_rel, b1, W2_root, W2_rel, b2, Wl, bl):
    raise NotImplementedError("write your pallas kernel here")


# speedup vs baseline: 3.8986x; 3.8986x over previous
import functools

import jax
import jax.numpy as jnp
from jax import lax
from jax.experimental import pallas as pl
from jax.experimental.pallas import tpu as pltpu
from jax.experimental.pallas import tpu_sc as plsc

_N = 100000
_E = 1600000

_CH = 80
_NCHUNK = _E // _CH
_TILES = 32
_CPT = _NCHUNK // _TILES
_RPT = _N // 16
_D = 16


def _make_agg():
  mesh = plsc.VectorSubcoreMesh(core_axis_name="c", subcore_axis_name="s")

  @functools.partial(
      pl.kernel,
      out_type=jax.ShapeDtypeStruct((2, _N, _D), jnp.float32),
      mesh=mesh,
      scratch_types=[
          pltpu.VMEM_SHARED((_N, _D), jnp.float32),
          pltpu.VMEM((_CH,), jnp.int32),
          pltpu.VMEM((_CH,), jnp.int32),
          pltpu.VMEM((_CH, _D), jnp.float32),
          pltpu.SemaphoreType.DMA,
      ],
      compiler_params=pltpu.CompilerParams(use_tc_tiling_on_sc=False),
  )
  def agg(h_hbm, g_hbm, zero_hbm, out_hbm, acc, src_v, dst_v, rows_v, sem):
    c = lax.axis_index("c")
    s = lax.axis_index("s")
    w = c * 16 + s
    base = w * _CPT

    pltpu.sync_copy(zero_hbm.at[pl.ds(s * _RPT, _RPT)],
                    acc.at[pl.ds(s * _RPT, _RPT)])
    plsc.subcore_barrier()

    def step(i, carry):
      chunk = base + i
      pltpu.sync_copy(g_hbm.at[0, chunk], src_v)
      pltpu.sync_copy(g_hbm.at[1, chunk], dst_v)
      pltpu.async_copy(h_hbm.at[src_v], rows_v, sem).wait()
      pltpu.sync_copy(rows_v, acc.at[dst_v], add=True)
      return carry

    lax.fori_loop(0, _CPT, step, 0)
    plsc.subcore_barrier()
    pltpu.sync_copy(acc.at[pl.ds(s * _RPT, _RPT)],
                    out_hbm.at[c].at[pl.ds(s * _RPT, _RPT)])

  return agg


_agg16 = _make_agg()

_BLK = 2000


def _tc_pre(feat, wa, wb, w_root, b):
  def body(x_ref, wa_ref, wb_ref, wroot_ref, b_ref, ha_ref, hb_ref, r_ref):
    x = x_ref[...]
    ha_ref[...] = jnp.dot(x, wa_ref[...], preferred_element_type=jnp.float32)
    hb_ref[...] = jnp.dot(x, wb_ref[...], preferred_element_type=jnp.float32)
    r_ref[...] = (jnp.dot(x, wroot_ref[...], preferred_element_type=jnp.float32)
                  + b_ref[...])

  return pl.pallas_call(
      body,
      grid=(_N // _BLK,),
      in_specs=[
          pl.BlockSpec((_BLK, 30), lambda i: (i, 0)),
          pl.BlockSpec((30, 16), lambda i: (0, 0)),
          pl.BlockSpec((30, 16), lambda i: (0, 0)),
          pl.BlockSpec((30, 20), lambda i: (0, 0)),
          pl.BlockSpec((1, 20), lambda i: (0, 0)),
      ],
      out_specs=[
          pl.BlockSpec((_BLK, 16), lambda i: (i, 0)),
          pl.BlockSpec((_BLK, 16), lambda i: (i, 0)),
          pl.BlockSpec((_BLK, 20), lambda i: (i, 0)),
      ],
      out_shape=[
          jax.ShapeDtypeStruct((_N, 16), jnp.float32),
          jax.ShapeDtypeStruct((_N, 16), jnp.float32),
          jax.ShapeDtypeStruct((_N, 20), jnp.float32),
      ],
  )(feat, wa, wb, w_root, b.reshape(1, 20))


def _tc_mid(r1, pa, pb, w_rel, w_root, b):
  def body(r_ref, pa_ref, pb_ref, wrel_ref, wroot_ref, b_ref, h_ref, r2_ref):
    agg_a = pa_ref[0] + pa_ref[1]
    agg_b = (pb_ref[0] + pb_ref[1])[:, :4]
    x = r_ref[...] + jnp.concatenate([agg_a, agg_b], axis=1)
    x = jnp.where(x > 0, x, 0.01 * x)
    x = x.reshape(_BLK, 10, 2).max(axis=2)
    h_ref[...] = jnp.dot(x, wrel_ref[...], preferred_element_type=jnp.float32)
    r2_ref[...] = (jnp.dot(x, wroot_ref[...], preferred_element_type=jnp.float32)
                   + b_ref[...])

  return pl.pallas_call(
      body,
      grid=(_N // _BLK,),
      in_specs=[
          pl.BlockSpec((_BLK, 20), lambda i: (i, 0)),
          pl.BlockSpec((2, _BLK, 16), lambda i: (0, i, 0)),
          pl.BlockSpec((2, _BLK, 16), lambda i: (0, i, 0)),
          pl.BlockSpec((10, 16), lambda i: (0, 0)),
          pl.BlockSpec((10, 5), lambda i: (0, 0)),
          pl.BlockSpec((1, 5), lambda i: (0, 0)),
      ],
      out_specs=[
          pl.BlockSpec((_BLK, 16), lambda i: (i, 0)),
          pl.BlockSpec((_BLK, 5), lambda i: (i, 0)),
      ],
      out_shape=[
          jax.ShapeDtypeStruct((_N, 16), jnp.float32),
          jax.ShapeDtypeStruct((_N, 5), jnp.float32),
      ],
  )(r1, pa, pb, w_rel, w_root, b.reshape(1, 5))


def _tc_post(r2, parts, wl, bl):
  def body(r_ref, p_ref, wl_ref, bl_ref, o_ref):
    x = r_ref[...] + p_ref[0, :, :5] + p_ref[1, :, :5]
    x = jnp.where(x > 0, x, 0.01 * x)
    o_ref[...] = (jnp.dot(x, wl_ref[...], preferred_element_type=jnp.float32)
                  + bl_ref[...])

  return pl.pallas_call(
      body,
      grid=(_N // _BLK,),
      in_specs=[
          pl.BlockSpec((_BLK, 5), lambda i: (i, 0)),
          pl.BlockSpec((2, _BLK, 16), lambda i: (0, i, 0)),
          pl.BlockSpec((5, 2), lambda i: (0, 0)),
          pl.BlockSpec((1, 2), lambda i: (0, 0)),
      ],
      out_specs=pl.BlockSpec((_BLK, 2), lambda i: (i, 0)),
      out_shape=jax.ShapeDtypeStruct((_N, 2), jnp.float32),
  )(r2, parts, wl, bl.reshape(1, 2))


@jax.jit
def kernel(g, edge_index, W1_root, W1_rel, b1, W2_root, W2_rel, b2, Wl, bl):
  feat = edge_index
  gc = g.reshape(2, _NCHUNK, _CH)
  zeros16 = jnp.zeros((_N, _D), jnp.float32)

  w1a = W1_rel[:, :16]
  w1b = jnp.pad(W1_rel[:, 16:], ((0, 0), (0, 12)))
  h1a, h1b, r1 = _tc_pre(feat, w1a, w1b, W1_root, b1)
  p1a = _agg16(h1a, gc, zeros16)
  p1b = _agg16(h1b, gc, zeros16)

  w2rel16 = jnp.pad(W2_rel, ((0, 0), (0, 11)))
  h2, r2 = _tc_mid(r1, p1a, p1b, w2rel16, W2_root, b2)
  p2 = _agg16(h2, gc, zeros16)

  return _tc_post(r2, p2, Wl, bl)


# speedup vs baseline: 3.9351x; 1.0094x over previous
import functools

import jax
import jax.numpy as jnp
from jax import lax
from jax.experimental import pallas as pl
from jax.experimental.pallas import tpu as pltpu
from jax.experimental.pallas import tpu_sc as plsc

_N = 100000
_E = 1600000

_CH = 80
_NCHUNK = _E // _CH
_TILES = 32
_CPT = _NCHUNK // _TILES
_RPT = _N // 16


def _make_agg(d):
  mesh = plsc.VectorSubcoreMesh(core_axis_name="c", subcore_axis_name="s")

  @functools.partial(
      pl.kernel,
      out_type=jax.ShapeDtypeStruct((2, _N, d), jnp.float32),
      mesh=mesh,
      scratch_types=[
          pltpu.VMEM_SHARED((_N, d), jnp.float32),
          pltpu.VMEM((_CH,), jnp.int32),
          pltpu.VMEM((_CH,), jnp.int32),
          pltpu.VMEM((_CH, d), jnp.float32),
          pltpu.SemaphoreType.DMA,
      ],
      compiler_params=pltpu.CompilerParams(use_tc_tiling_on_sc=False),
  )
  def agg(h_hbm, g_hbm, zero_hbm, out_hbm, acc, src_v, dst_v, rows_v, sem):
    c = lax.axis_index("c")
    s = lax.axis_index("s")
    w = c * 16 + s
    base = w * _CPT

    pltpu.sync_copy(zero_hbm.at[pl.ds(s * _RPT, _RPT)],
                    acc.at[pl.ds(s * _RPT, _RPT)])
    plsc.subcore_barrier()

    def step(i, carry):
      chunk = base + i
      pltpu.sync_copy(g_hbm.at[0, chunk], src_v)
      pltpu.sync_copy(g_hbm.at[1, chunk], dst_v)
      pltpu.async_copy(h_hbm.at[src_v], rows_v, sem).wait()
      pltpu.sync_copy(rows_v, acc.at[dst_v], add=True)
      return carry

    lax.fori_loop(0, _CPT, step, 0)
    plsc.subcore_barrier()
    pltpu.sync_copy(acc.at[pl.ds(s * _RPT, _RPT)],
                    out_hbm.at[c].at[pl.ds(s * _RPT, _RPT)])

  return agg


_agg16 = _make_agg(16)
_agg8 = _make_agg(8)

_BLK = 2000


def _tc_pre(feat, wa, wb, w_root, b):
  def body(x_ref, wa_ref, wb_ref, wroot_ref, b_ref, ha_ref, hb_ref, r_ref):
    x = x_ref[...]
    ha_ref[...] = jnp.dot(x, wa_ref[...], preferred_element_type=jnp.float32)
    hb_ref[...] = jnp.dot(x, wb_ref[...], preferred_element_type=jnp.float32)
    r_ref[...] = (jnp.dot(x, wroot_ref[...], preferred_element_type=jnp.float32)
                  + b_ref[...])

  return pl.pallas_call(
      body,
      grid=(_N // _BLK,),
      in_specs=[
          pl.BlockSpec((_BLK, 30), lambda i: (i, 0)),
          pl.BlockSpec((30, 16), lambda i: (0, 0)),
          pl.BlockSpec((30, 8), lambda i: (0, 0)),
          pl.BlockSpec((30, 20), lambda i: (0, 0)),
          pl.BlockSpec((1, 20), lambda i: (0, 0)),
      ],
      out_specs=[
          pl.BlockSpec((_BLK, 16), lambda i: (i, 0)),
          pl.BlockSpec((_BLK, 8), lambda i: (i, 0)),
          pl.BlockSpec((_BLK, 20), lambda i: (i, 0)),
      ],
      out_shape=[
          jax.ShapeDtypeStruct((_N, 16), jnp.float32),
          jax.ShapeDtypeStruct((_N, 8), jnp.float32),
          jax.ShapeDtypeStruct((_N, 20), jnp.float32),
      ],
  )(feat, wa, wb, w_root, b.reshape(1, 20))


def _tc_mid(r1, pa, pb, w_rel, w_root, b):
  def body(r_ref, pa_ref, pb_ref, wrel_ref, wroot_ref, b_ref, h_ref, r2_ref):
    agg_a = pa_ref[0] + pa_ref[1]
    agg_b = (pb_ref[0] + pb_ref[1])[:, :4]
    x = r_ref[...] + jnp.concatenate([agg_a, agg_b], axis=1)
    x = jnp.where(x > 0, x, 0.01 * x)
    x = x.reshape(_BLK, 10, 2).max(axis=2)
    h_ref[...] = jnp.dot(x, wrel_ref[...], preferred_element_type=jnp.float32)
    r2_ref[...] = (jnp.dot(x, wroot_ref[...], preferred_element_type=jnp.float32)
                   + b_ref[...])

  return pl.pallas_call(
      body,
      grid=(_N // _BLK,),
      in_specs=[
          pl.BlockSpec((_BLK, 20), lambda i: (i, 0)),
          pl.BlockSpec((2, _BLK, 16), lambda i: (0, i, 0)),
          pl.BlockSpec((2, _BLK, 8), lambda i: (0, i, 0)),
          pl.BlockSpec((10, 8), lambda i: (0, 0)),
          pl.BlockSpec((10, 5), lambda i: (0, 0)),
          pl.BlockSpec((1, 5), lambda i: (0, 0)),
      ],
      out_specs=[
          pl.BlockSpec((_BLK, 8), lambda i: (i, 0)),
          pl.BlockSpec((_BLK, 5), lambda i: (i, 0)),
      ],
      out_shape=[
          jax.ShapeDtypeStruct((_N, 8), jnp.float32),
          jax.ShapeDtypeStruct((_N, 5), jnp.float32),
      ],
  )(r1, pa, pb, w_rel, w_root, b.reshape(1, 5))


def _tc_post(r2, parts, wl, bl):
  def body(r_ref, p_ref, wl_ref, bl_ref, o_ref):
    x = r_ref[...] + p_ref[0, :, :5] + p_ref[1, :, :5]
    x = jnp.where(x > 0, x, 0.01 * x)
    o_ref[...] = (jnp.dot(x, wl_ref[...], preferred_element_type=jnp.float32)
                  + bl_ref[...])

  return pl.pallas_call(
      body,
      grid=(_N // _BLK,),
      in_specs=[
          pl.BlockSpec((_BLK, 5), lambda i: (i, 0)),
          pl.BlockSpec((2, _BLK, 8), lambda i: (0, i, 0)),
          pl.BlockSpec((5, 2), lambda i: (0, 0)),
          pl.BlockSpec((1, 2), lambda i: (0, 0)),
      ],
      out_specs=pl.BlockSpec((_BLK, 2), lambda i: (i, 0)),
      out_shape=jax.ShapeDtypeStruct((_N, 2), jnp.float32),
  )(r2, parts, wl, bl.reshape(1, 2))


@jax.jit
def kernel(g, edge_index, W1_root, W1_rel, b1, W2_root, W2_rel, b2, Wl, bl):
  feat = edge_index
  gc = g.reshape(2, _NCHUNK, _CH)
  zeros16 = jnp.zeros((_N, 16), jnp.float32)
  zeros8 = jnp.zeros((_N, 8), jnp.float32)

  w1a = W1_rel[:, :16]
  w1b = jnp.pad(W1_rel[:, 16:], ((0, 0), (0, 4)))
  h1a, h1b, r1 = _tc_pre(feat, w1a, w1b, W1_root, b1)
  p1a = _agg16(h1a, gc, zeros16)
  p1b = _agg8(h1b, gc, zeros8)

  w2rel8 = jnp.pad(W2_rel, ((0, 0), (0, 3)))
  h2, r2 = _tc_mid(r1, p1a, p1b, w2rel8, W2_root, b2)
  p2 = _agg8(h2, gc, zeros8)

  return _tc_post(r2, p2, Wl, bl)


# speedup vs baseline: 12.4678x; 3.1684x over previous
import functools

import jax
import jax.numpy as jnp
from jax import lax
from jax.experimental import pallas as pl
from jax.experimental.pallas import tpu as pltpu
from jax.experimental.pallas import tpu_sc as plsc

_N = 100000
_E = 1600000

_CH = 80
_NCHUNK = _E // _CH
_TILES = 32
_CPT = _NCHUNK // _TILES
_RPT = _N // 16
_K = 5
_NB = _CPT // _K


def _make_agg(d):
  mesh = plsc.VectorSubcoreMesh(core_axis_name="c", subcore_axis_name="s")

  @functools.partial(
      pl.kernel,
      out_type=jax.ShapeDtypeStruct((2, _N, d), jnp.float32),
      mesh=mesh,
      scratch_types=[
          pltpu.VMEM_SHARED((_N, d), jnp.float32),
          pltpu.VMEM((2, _K, _CH), jnp.int32),
          pltpu.VMEM((3, _K, _CH), jnp.int32),
          pltpu.VMEM((_K, _CH, d), jnp.float32),
          pltpu.SemaphoreType.DMA,
      ] + [pltpu.SemaphoreType.DMA] * (2 * _K),
      compiler_params=pltpu.CompilerParams(use_tc_tiling_on_sc=False),
  )
  def agg(h_hbm, g_hbm, zero_hbm, out_hbm, acc, sstage, dstage, rows,
          isem, *sems):
    gsem = sems[:_K]
    ssem = sems[_K:]
    c = lax.axis_index("c")
    s = lax.axis_index("s")
    w = c * 16 + s
    base = w * _CPT

    pltpu.sync_copy(zero_hbm.at[pl.ds(s * _RPT, _RPT)],
                    acc.at[pl.ds(s * _RPT, _RPT)])
    plsc.subcore_barrier()

    def idx_copies(bn):
      jb = base + bn * _K
      return (
          pltpu.make_async_copy(g_hbm.at[0, pl.ds(jb, _K)],
                                sstage.at[lax.rem(bn, 2)], isem),
          pltpu.make_async_copy(g_hbm.at[1, pl.ds(jb, _K)],
                                dstage.at[lax.rem(bn, 3)], isem),
      )

    def prefetch(bn):
      for cp in idx_copies(bn):
        cp.start()

    def wait_prefetch(bn):
      for cp in idx_copies(bn):
        cp.wait()

    prefetch(0)

    def batch(b, carry):
      bi2 = lax.rem(b, 2)
      bi3 = lax.rem(b, 3)
      bp3 = lax.rem(b + 2, 3)
      wait_prefetch(b)

      @pl.when(b + 1 < _NB)
      def _():
        prefetch(b + 1)

      for j in range(_K):
        @pl.when(b > 0)
        def _():
          pltpu.make_async_copy(rows.at[j], acc.at[dstage.at[bp3, j]],
                                ssem[j]).wait()
        pltpu.async_copy(h_hbm.at[sstage.at[bi2, j]], rows.at[j], gsem[j])
      for j in range(_K):
        pltpu.make_async_copy(h_hbm.at[sstage.at[bi2, j]], rows.at[j],
                              gsem[j]).wait()
        pltpu.async_copy(rows.at[j], acc.at[dstage.at[bi3, j]], ssem[j],
                         add=True)
      return carry

    lax.fori_loop(0, _NB, batch, 0)
    for j in range(_K):
      pltpu.make_async_copy(rows.at[j], acc.at[dstage.at[(_NB - 1) % 3, j]],
                            ssem[j]).wait()
    plsc.subcore_barrier()
    pltpu.sync_copy(acc.at[pl.ds(s * _RPT, _RPT)],
                    out_hbm.at[c].at[pl.ds(s * _RPT, _RPT)])

  return agg


_agg16 = _make_agg(16)
_agg8 = _make_agg(8)

_BLK = 2000


def _tc_pre(feat, wa, wb, w_root, b):
  def body(x_ref, wa_ref, wb_ref, wroot_ref, b_ref, ha_ref, hb_ref, r_ref):
    x = x_ref[...]
    ha_ref[...] = jnp.dot(x, wa_ref[...], preferred_element_type=jnp.float32)
    hb_ref[...] = jnp.dot(x, wb_ref[...], preferred_element_type=jnp.float32)
    r_ref[...] = (jnp.dot(x, wroot_ref[...], preferred_element_type=jnp.float32)
                  + b_ref[...])

  return pl.pallas_call(
      body,
      grid=(_N // _BLK,),
      in_specs=[
          pl.BlockSpec((_BLK, 30), lambda i: (i, 0)),
          pl.BlockSpec((30, 16), lambda i: (0, 0)),
          pl.BlockSpec((30, 8), lambda i: (0, 0)),
          pl.BlockSpec((30, 20), lambda i: (0, 0)),
          pl.BlockSpec((1, 20), lambda i: (0, 0)),
      ],
      out_specs=[
          pl.BlockSpec((_BLK, 16), lambda i: (i, 0)),
          pl.BlockSpec((_BLK, 8), lambda i: (i, 0)),
          pl.BlockSpec((_BLK, 20), lambda i: (i, 0)),
      ],
      out_shape=[
          jax.ShapeDtypeStruct((_N, 16), jnp.float32),
          jax.ShapeDtypeStruct((_N, 8), jnp.float32),
          jax.ShapeDtypeStruct((_N, 20), jnp.float32),
      ],
  )(feat, wa, wb, w_root, b.reshape(1, 20))


def _tc_mid(r1, pa, pb, w_rel, w_root, b):
  def body(r_ref, pa_ref, pb_ref, wrel_ref, wroot_ref, b_ref, h_ref, r2_ref):
    agg_a = pa_ref[0] + pa_ref[1]
    agg_b = (pb_ref[0] + pb_ref[1])[:, :4]
    x = r_ref[...] + jnp.concatenate([agg_a, agg_b], axis=1)
    x = jnp.where(x > 0, x, 0.01 * x)
    x = x.reshape(_BLK, 10, 2).max(axis=2)
    h_ref[...] = jnp.dot(x, wrel_ref[...], preferred_element_type=jnp.float32)
    r2_ref[...] = (jnp.dot(x, wroot_ref[...], preferred_element_type=jnp.float32)
                   + b_ref[...])

  return pl.pallas_call(
      body,
      grid=(_N // _BLK,),
      in_specs=[
          pl.BlockSpec((_BLK, 20), lambda i: (i, 0)),
          pl.BlockSpec((2, _BLK, 16), lambda i: (0, i, 0)),
          pl.BlockSpec((2, _BLK, 8), lambda i: (0, i, 0)),
          pl.BlockSpec((10, 8), lambda i: (0, 0)),
          pl.BlockSpec((10, 5), lambda i: (0, 0)),
          pl.BlockSpec((1, 5), lambda i: (0, 0)),
      ],
      out_specs=[
          pl.BlockSpec((_BLK, 8), lambda i: (i, 0)),
          pl.BlockSpec((_BLK, 5), lambda i: (i, 0)),
      ],
      out_shape=[
          jax.ShapeDtypeStruct((_N, 8), jnp.float32),
          jax.ShapeDtypeStruct((_N, 5), jnp.float32),
      ],
  )(r1, pa, pb, w_rel, w_root, b.reshape(1, 5))


def _tc_post(r2, parts, wl, bl):
  def body(r_ref, p_ref, wl_ref, bl_ref, o_ref):
    x = r_ref[...] + p_ref[0, :, :5] + p_ref[1, :, :5]
    x = jnp.where(x > 0, x, 0.01 * x)
    o_ref[...] = (jnp.dot(x, wl_ref[...], preferred_element_type=jnp.float32)
                  + bl_ref[...])

  return pl.pallas_call(
      body,
      grid=(_N // _BLK,),
      in_specs=[
          pl.BlockSpec((_BLK, 5), lambda i: (i, 0)),
          pl.BlockSpec((2, _BLK, 8), lambda i: (0, i, 0)),
          pl.BlockSpec((5, 2), lambda i: (0, 0)),
          pl.BlockSpec((1, 2), lambda i: (0, 0)),
      ],
      out_specs=pl.BlockSpec((_BLK, 2), lambda i: (i, 0)),
      out_shape=jax.ShapeDtypeStruct((_N, 2), jnp.float32),
  )(r2, parts, wl, bl.reshape(1, 2))


@jax.jit
def kernel(g, edge_index, W1_root, W1_rel, b1, W2_root, W2_rel, b2, Wl, bl):
  feat = edge_index
  gc = g.reshape(2, _NCHUNK, _CH)
  zeros16 = jnp.zeros((_N, 16), jnp.float32)
  zeros8 = jnp.zeros((_N, 8), jnp.float32)

  w1a = W1_rel[:, :16]
  w1b = jnp.pad(W1_rel[:, 16:], ((0, 0), (0, 4)))
  h1a, h1b, r1 = _tc_pre(feat, w1a, w1b, W1_root, b1)
  p1a = _agg16(h1a, gc, zeros16)
  p1b = _agg8(h1b, gc, zeros8)

  w2rel8 = jnp.pad(W2_rel, ((0, 0), (0, 3)))
  h2, r2 = _tc_mid(r1, p1a, p1b, w2rel8, W2_root, b2)
  p2 = _agg8(h2, gc, zeros8)

  return _tc_post(r2, p2, Wl, bl)
